# Initial kernel scaffold; baseline (speedup 1.0000x reference)
import jax
import jax.numpy as jnp
from jax.experimental import pallas as pl


def kernel(x, gain, bias, log_mix, log_momentum, log_scale):
    raise NotImplementedError("write your pallas kernel here")


# speedup vs baseline: 10.4794x; 10.4794x over previous
import functools
import math

import jax
import jax.numpy as jnp
from jax.experimental import pallas as pl
from jax.experimental.pallas import tpu as pltpu

K = 8
R = 3
NEG = -1e38
NEG_GUARD = -1e37


def _round_body(r, is_last, BT, T, D, NI,
                params_ref, h_ref, x_ref, gain_ref, bias_ref,
                out_ref, s_ref, a_ref, m_ref):
    i = pl.program_id(1)
    mix = params_ref[r]
    momentum = params_ref[R]
    scale = params_ref[R + 1]

    q = h_ref[0, pl.ds(i * BT, BT), :]

    s_ref[...] = jnp.full((BT, T), NEG, jnp.float32)
    a_ref[...] = jnp.zeros((BT, T), jnp.float32)

    def fill(j, carry):
        kblk = h_ref[0, pl.ds(j * BT, BT), :]
        s_ref[:, pl.ds(j * BT, BT)] = jax.lax.dot_general(
            q, kblk, (((1,), (1,)), ((), ())),
            preferred_element_type=jnp.float32)
        return carry

    jax.lax.fori_loop(0, i + 1, fill, 0)

    rows = i * BT + jax.lax.broadcasted_iota(jnp.int32, (BT, T), 0)
    cols = jax.lax.broadcasted_iota(jnp.int32, (BT, T), 1)
    s_ref[...] = jnp.where(cols <= rows, s_ref[...], NEG)

    for _ in range(K):
        s = s_ref[...]
        m = jnp.max(s, axis=1, keepdims=True)
        hit = (s == m) & (m > NEG_GUARD)
        idxsel = jnp.min(jnp.where(hit, cols, T), axis=1, keepdims=True)
        pick = cols == idxsel
        a_ref[...] += pick.astype(jnp.float32)
        s_ref[...] = jnp.where(pick, NEG, s)

    m_ref[...] = jnp.zeros((BT, D), jnp.float32)

    def agg(j, carry):
        ablk = a_ref[:, pl.ds(j * BT, BT)]
        hblk = h_ref[0, pl.ds(j * BT, BT), :]
        m_ref[...] += jax.lax.dot_general(
            ablk, hblk, (((1,), (0,)), ((), ())),
            preferred_element_type=jnp.float32)
        return carry

    jax.lax.fori_loop(0, i + 1, agg, 0)

    row1 = i * BT + jax.lax.broadcasted_iota(jnp.int32, (BT, 1), 0)
    deg = jnp.minimum(row1.astype(jnp.float32) + 1.0, float(K))
    msg = m_ref[...] / deg

    blended = mix * q + (1.0 - mix) * msg
    gb = blended * gain_ref[0] + bias_ref[0]
    act = gb * 0.5 * (1.0 + jax.lax.erf(gb * (1.0 / math.sqrt(2.0))))
    hn = momentum * q + (1.0 - momentum) * act
    if is_last:
        out_ref[0] = (hn - x_ref[0, pl.ds(i * BT, BT), :]) * scale
    else:
        out_ref[0] = hn


def _round_call(r, is_last, h, x, gain_r, bias_r, params, BT=256):
    B, T, D = h.shape
    NI = T // BT
    body = functools.partial(_round_body, r, is_last, BT, T, D, NI)
    return pl.pallas_call(
        body,
        grid=(B, NI),
        in_specs=[
            pl.BlockSpec(memory_space=pltpu.SMEM),
            pl.BlockSpec((1, T, D), lambda b, i: (b, 0, 0)),
            pl.BlockSpec((1, T, D), lambda b, i: (b, 0, 0)),
            pl.BlockSpec((1, D), lambda b, i: (0, 0)),
            pl.BlockSpec((1, D), lambda b, i: (0, 0)),
        ],
        out_specs=pl.BlockSpec((1, BT, D), lambda b, i: (b, i, 0)),
        out_shape=jax.ShapeDtypeStruct((B, T, D), jnp.float32),
        scratch_shapes=[
            pltpu.VMEM((BT, T), jnp.float32),
            pltpu.VMEM((BT, T), jnp.float32),
            pltpu.VMEM((BT, D), jnp.float32),
        ],
    )(params, h, x, gain_r, bias_r)


def kernel(x, gain, bias, log_mix, log_momentum, log_scale):
    B, T, D = x.shape
    momentum = jax.nn.sigmoid(log_momentum)
    scale = jax.nn.softplus(log_scale) + 0.01
    mix = jax.nn.sigmoid(log_mix)
    params = jnp.concatenate(
        [mix.astype(jnp.float32),
         jnp.stack([momentum, scale]).astype(jnp.float32)])
    h = x
    for r in range(R):
        h = _round_call(r, r == R - 1, h, x,
                        gain[r].reshape(1, D), bias[r].reshape(1, D), params)
    return h
